# Initial kernel scaffold; baseline (speedup 1.0000x reference)
import jax
import jax.numpy as jnp
from jax.experimental import pallas as pl


def kernel(x, W_inp, b_inp, W_out, b_out, W_read, b_read):
    raise NotImplementedError("write your pallas kernel here")


# speedup vs baseline: 197.9247x; 197.9247x over previous
import math

import jax
import jax.numpy as jnp
import numpy as np
from jax.experimental import pallas as pl
from jax.experimental.pallas import tpu as pltpu

_M, _D, _N, _R = 16384, 64, 8, 8
_T, _B = 8, 8
_S = 0.5
_PROBS = [0.7, 0.3, 0.5]
_WIN = 2 * _R + 1


def _ring_plan():
    step = int(_M * ((math.sqrt(5) - 1) / 2))
    ptr = np.zeros(_N, np.float32)
    centers = np.zeros((_T, _N), np.int64)
    for t in range(_T):
        for i in range(_N):
            c = int(np.clip(np.int32(ptr[i]), 0, _M - 1))
            centers[t, i] = c
            jump = np.float32((c + step + i) % _M)
            walk = np.float32((ptr[i] + np.float32(1.0)) % _M)
            p = np.float32(_PROBS[i % 3])
            q = np.float32(1.0 - _PROBS[i % 3])
            ptr[i] = np.float32(p * jump) + np.float32(q * walk)
    offs = np.arange(-_R, _R + 1)
    wins = (centers[:, :, None] + offs[None, None, :]) % _M
    slots = np.unique(wins)
    lut = {int(s): k for k, s in enumerate(slots)}
    segs = []
    for t in range(_T):
        row = []
        for i in range(_N):
            ks = sorted(lut[int(s)] for s in wins[t, i])
            runs = []
            a = prev = ks[0]
            for k in ks[1:]:
                if k == prev + 1:
                    prev = k
                else:
                    runs.append((a, prev - a + 1))
                    a = prev = k
            runs.append((a, prev - a + 1))
            row.append(runs)
        segs.append(row)
    return len(slots), segs


_K, _SEGS = _ring_plan()
_DN = (((1,), (1,)), ((), ()))


def _body(x_ref, wi_ref, bi_ref, wr_ref, br_ref, wo_ref, bo_ref,
          out_ref, ring_ref):
    f32 = jnp.float32
    ring_ref[:] = jnp.zeros((_K, _B, _D), f32)
    inp = jax.lax.dot_general(x_ref[:], wi_ref[:], _DN,
                              preferred_element_type=f32) + bi_ref[:]
    hidden = [jnp.zeros((_B, _D), f32) for _ in range(_N)]
    means = []
    for t in range(_T):
        inp_t = inp[t * _B:(t + 1) * _B]
        for i in range(_N):
            h = hidden[i] + inp_t
            acc = None
            for (a, ln) in _SEGS[t][i]:
                s = jnp.sum(ring_ref[a:a + ln], axis=0)
                acc = s if acc is None else acc + s
            read = acc * (1.0 / _WIN)
            rv = jax.lax.dot_general(read, wr_ref[i], _DN,
                                     preferred_element_type=f32)
            h = h + _S * (rv + br_ref[i:i + 1, :])
            hidden[i] = h
            v = h * (1.0 / _WIN)
            for (a, ln) in _SEGS[t][i]:
                ring_ref[a:a + ln] = ring_ref[a:a + ln] + v[None]
        hs = hidden[0]
        for i in range(1, _N):
            hs = hs + hidden[i]
        means.append(hs * (1.0 / _N))
    mean = jnp.concatenate(means, axis=0)
    out_ref[:] = jax.lax.dot_general(mean, wo_ref[:], _DN,
                                     preferred_element_type=f32) + bo_ref[:]


def kernel(x, W_inp, b_inp, W_out, b_out, W_read, b_read):
    bb, tt, feat = x.shape
    xt = x.transpose(1, 0, 2).reshape(tt * bb, feat)
    out = pl.pallas_call(
        _body,
        out_shape=jax.ShapeDtypeStruct((_T * _B, feat), jnp.float32),
        scratch_shapes=[pltpu.VMEM((_K, _B, _D), jnp.float32)],
    )(xt, W_inp, b_inp.reshape(1, -1), W_read, b_read,
      W_out, b_out.reshape(1, -1))
    return out.reshape(tt, bb, feat).transpose(1, 0, 2)
